# Initial kernel scaffold; baseline (speedup 1.0000x reference)
import jax
import jax.numpy as jnp
from jax.experimental import pallas as pl


def kernel(x, edge_index, W1, b1, W2, b2, Wl, bl):
    raise NotImplementedError("write your pallas kernel here")


# speedup vs baseline: 16.1815x; 16.1815x over previous
import jax
import jax.numpy as jnp
from jax import lax
from jax.experimental import pallas as pl
from jax.experimental.pallas import tpu as pltpu
from jax.experimental.pallas import tpu_sc as plsc

N = 10000
E = 320000
F = 128
H1 = 128
H2 = 16
C = 9

B = 80
NB = E // B
NB_PAD = 4096
NC = 2
NS = 16
NW = NC * NS
RPW = NB_PAD // NW
KG = 8
GROUPS = RPW // KG
N_PAD = 10240
RPT = N_PAD // NS

_MESH = dict(core_axis_name="c", subcore_axis_name="s")


def _sc_agg(D, feature_split):
    groups = (NB_PAD // NS if feature_split else RPW) // KG

    def body(g_hbm, src_hbm, dst_hbm, z_hbm, out_hbm, src_v, dst_v, rows_v,
             acc, sem):
        c = lax.axis_index("c")
        s = lax.axis_index("s")
        pltpu.sync_copy(z_hbm.at[pl.ds(s * RPT, RPT)],
                        acc.at[pl.ds(s * RPT, RPT)])
        plsc.subcore_barrier()
        if feature_split:
            base = s * (NB_PAD // NS)
            gsrc = g_hbm.at[c]
        else:
            base = (s * NC + c) * RPW
            gsrc = g_hbm

        @pl.loop(0, groups)
        def _(i):
            r0 = base + i * KG
            pltpu.sync_copy(src_hbm.at[pl.ds(r0, KG)], src_v)
            pltpu.sync_copy(dst_hbm.at[pl.ds(r0, KG)], dst_v)
            descs = [
                pltpu.async_copy(gsrc.at[src_v.at[j]], rows_v.at[j], sem)
                for j in range(KG)
            ]
            for d in descs:
                d.wait()
            for j in range(KG):
                pltpu.sync_copy(rows_v.at[j], acc.at[dst_v.at[j]], add=True)

        plsc.subcore_barrier()
        pltpu.sync_copy(acc.at[pl.ds(s * RPT, RPT)],
                        out_hbm.at[c].at[pl.ds(s * RPT, RPT)])

    return pl.kernel(
        body,
        out_type=jax.ShapeDtypeStruct((NC, N_PAD, D), jnp.float32),
        mesh=plsc.VectorSubcoreMesh(**_MESH),
        compiler_params=pltpu.CompilerParams(use_tc_tiling_on_sc=False),
        scratch_types=[
            pltpu.VMEM((KG, B), jnp.int32),
            pltpu.VMEM((KG, B), jnp.int32),
            pltpu.VMEM((KG, B, D), jnp.float32),
            pltpu.VMEM_SHARED((N_PAD, D), jnp.float32),
            pltpu.SemaphoreType.DMA,
        ],
    )


def _sc_deg_body(dst_hbm, z_hbm, out_hbm, dst_v, ones_v, acc, sem):
    c = lax.axis_index("c")
    s = lax.axis_index("s")
    wid = s * NC + c
    for r in range(B):
        ones_v[r] = jnp.full((16,), 1.0, jnp.float32)
    pltpu.sync_copy(z_hbm.at[pl.ds(s * RPT, RPT)], acc.at[pl.ds(s * RPT, RPT)])
    plsc.subcore_barrier()
    base = wid * RPW

    @pl.loop(0, GROUPS)
    def _(i):
        r0 = base + i * KG
        pltpu.sync_copy(dst_hbm.at[pl.ds(r0, KG)], dst_v)
        for j in range(KG):
            pltpu.sync_copy(ones_v, acc.at[dst_v.at[j]], add=True)

    plsc.subcore_barrier()
    pltpu.sync_copy(acc.at[pl.ds(s * RPT, RPT)],
                    out_hbm.at[c].at[pl.ds(s * RPT, RPT)])


_sc_deg = pl.kernel(
    _sc_deg_body,
    out_type=jax.ShapeDtypeStruct((NC, N_PAD, 16), jnp.float32),
    mesh=plsc.VectorSubcoreMesh(**_MESH),
    compiler_params=pltpu.CompilerParams(use_tc_tiling_on_sc=False),
    scratch_types=[
        pltpu.VMEM((KG, B), jnp.int32),
        pltpu.VMEM((B, 16), jnp.float32),
        pltpu.VMEM_SHARED((N_PAD, 16), jnp.float32),
        pltpu.SemaphoreType.DMA,
    ],
)


M_BLK = 400
GRID = N // M_BLK


def _tc1_body(x_ref, w1_ref, hist_ref, g1_ref, dinv_ref):
    h = jnp.dot(x_ref[...], w1_ref[...], preferred_element_type=jnp.float32)
    deg = hist_ref[0, :, 0:1] + hist_ref[1, :, 0:1] + 1.0
    dinv = lax.rsqrt(deg)
    g = h * dinv
    g1_ref[0] = g[:, :64]
    g1_ref[1] = g[:, 64:]
    dinv_ref[...] = dinv


def _tc2_body(s1_ref, g1_ref, dinv_ref, b1_ref, w2_ref, g2_ref):
    dinv = dinv_ref[...]
    h = jnp.concatenate([s1_ref[0] + g1_ref[0], s1_ref[1] + g1_ref[1]],
                        axis=1)
    h = jnp.maximum(h * dinv + b1_ref[...], 0.0)
    g2_ref[...] = jnp.dot(h, w2_ref[...],
                          preferred_element_type=jnp.float32) * dinv


def _tc3_body(s2_ref, g2_ref, dinv_ref, b2_ref, wl_ref, bl_ref, out_ref):
    dinv = dinv_ref[...]
    h = (s2_ref[0] + s2_ref[1] + g2_ref[...]) * dinv + b2_ref[...]
    h = jnp.maximum(h, 0.0)
    out_ref[...] = jnp.dot(h, wl_ref[...],
                           preferred_element_type=jnp.float32) + bl_ref[...]


def _row_blk(d):
    return pl.BlockSpec((M_BLK, d), lambda i: (i, 0))


def _pair_blk(d):
    return pl.BlockSpec((2, M_BLK, d), lambda i: (0, i, 0))


def _full_blk(shape):
    return pl.BlockSpec(shape, lambda i: tuple(0 for _ in shape))


_tc1 = pl.pallas_call(
    _tc1_body,
    grid=(GRID,),
    in_specs=[_row_blk(F), _full_blk((F, H1)), _pair_blk(16)],
    out_specs=[_pair_blk(64), _row_blk(1)],
    out_shape=[
        jax.ShapeDtypeStruct((2, N, 64), jnp.float32),
        jax.ShapeDtypeStruct((N, 1), jnp.float32),
    ],
)

_tc2 = pl.pallas_call(
    _tc2_body,
    grid=(GRID,),
    in_specs=[_pair_blk(64), _pair_blk(64), _row_blk(1), _full_blk((1, H1)),
              _full_blk((H1, H2))],
    out_specs=_row_blk(H2),
    out_shape=jax.ShapeDtypeStruct((N, H2), jnp.float32),
)

_tc3 = pl.pallas_call(
    _tc3_body,
    grid=(GRID,),
    in_specs=[_pair_blk(H2), _row_blk(H2), _row_blk(1), _full_blk((1, H2)),
              _full_blk((H2, C)), _full_blk((1, C))],
    out_specs=_row_blk(C),
    out_shape=jax.ShapeDtypeStruct((N, C), jnp.float32),
)

_agg1 = _sc_agg(64, feature_split=True)
_agg2 = _sc_agg(H2, feature_split=False)


def kernel(x, edge_index, W1, b1, W2, b2, Wl, bl):
    ei = edge_index.astype(jnp.int32)
    pad = NB_PAD - NB
    src2 = jnp.concatenate(
        [ei[0].reshape(NB, B), jnp.zeros((pad, B), jnp.int32)])
    dst2 = jnp.concatenate(
        [ei[1].reshape(NB, B), jnp.full((pad, B), N, jnp.int32)])
    z64 = jnp.zeros((N_PAD, 64), jnp.float32)
    z16 = jnp.zeros((N_PAD, 16), jnp.float32)

    hist = _sc_deg(dst2, z16)
    g1, dinv = _tc1(x, W1, hist)
    s1 = _agg1(g1, src2, dst2, z64)
    g2 = _tc2(s1, g1, dinv, b1.reshape(1, H1), W2)
    s2 = _agg2(g2, src2, dst2, z16)
    out = _tc3(s2, g2, dinv, b2.reshape(1, H2), Wl, bl.reshape(1, C))
    return out
